# baseline (speedup 1.0000x reference)
import functools

import jax
import jax.numpy as jnp
from jax import lax
from jax.experimental import pallas as pl
from jax.experimental.pallas import tpu as pltpu
from jax.experimental.pallas import tpu_sc as plsc

START_MIN, START_MAX = 1, 2
END_MIN, END_MAX = 3, 4
N_SEG = 16
LANES = 16


def _sc_gather_kernel(x2d, ids, out0, out1, ids_v, idx_v, rows_v, tmp_v, sem,
                      *, b_total, s_len, d):
    wid = lax.axis_index("s")
    b = wid % b_total
    which = wid // b_total

    @pl.when(which < 2)
    def _():
        pltpu.sync_copy(ids.at[b], ids_v)
        lo = jnp.where(which == 0, START_MIN, END_MIN)
        hi = jnp.where(which == 0, START_MAX, END_MAX)
        lane = lax.iota(jnp.int32, LANES)

        def body(ci, cnt):
            v = ids_v[pl.ds(ci * LANES, LANES)]
            m = (v >= lo) & (v <= hi)

            def matched(cnt):
                cur = jnp.where(m, 1, 0)
                for sh in (1, 2, 4, 8):
                    tmp_v[...] = cur
                    g = plsc.load_gather(tmp_v, [jnp.maximum(lane - sh, 0)])
                    cur = cur + jnp.where(lane >= sh, g, 0)
                ranks = jnp.clip(cnt + cur - 1, 0, N_SEG - 1)
                pos = b * s_len + ci * LANES + lane
                plsc.store_scatter(idx_v, [ranks], pos, mask=m)
                return cnt + plsc.all_reduce_population_count(m)

            return lax.cond(jnp.any(m), matched, lambda c: c, cnt)

        lax.fori_loop(0, s_len // LANES, body,
                      jnp.zeros((LANES,), jnp.int32))
        pltpu.async_copy(x2d.at[idx_v], rows_v, sem).wait()

        @pl.when(which == 0)
        def _():
            pltpu.sync_copy(rows_v, out0.at[b])

        @pl.when(which == 1)
        def _():
            pltpu.sync_copy(rows_v, out1.at[b])


def _sc_gather(x, input_ids):
    b, s, d = x.shape
    x2d = x.reshape(b * s, d)
    mesh = plsc.VectorSubcoreMesh(
        core_axis_name="c", subcore_axis_name="s", num_cores=1)
    kern = functools.partial(
        pl.kernel,
        out_type=[
            jax.ShapeDtypeStruct((b, N_SEG, d), jnp.float32),
            jax.ShapeDtypeStruct((b, N_SEG, d), jnp.float32),
        ],
        mesh=mesh,
        scratch_types=[
            pltpu.VMEM((s,), jnp.int32),
            pltpu.VMEM((N_SEG,), jnp.int32),
            pltpu.VMEM((N_SEG, d), jnp.float32),
            pltpu.VMEM((LANES,), jnp.int32),
            pltpu.SemaphoreType.DMA,
        ],
        compiler_params=pltpu.CompilerParams(
            needs_layout_passes=False, skip_device_barrier=True),
        cost_estimate=pl.CostEstimate(
            flops=0,
            bytes_accessed=2 * b * N_SEG * d * 4 + b * s * 4,
            transcendentals=0,
        ),
    )(functools.partial(_sc_gather_kernel, b_total=b, s_len=s, d=d))
    return kern(x2d, input_ids)


def _incl_cumsum_lanes(a):
    s = a.shape[1]
    sh = 1
    while sh < s:
        shifted = jnp.concatenate(
            [jnp.zeros((1, sh), a.dtype), a[:, : s - sh]], axis=1)
        a = a + shifted
        sh *= 2
    return a


def _tc_mean_kernel(ids_ref, x_ref, o_ref):
    ids = ids_ref[0]
    s = ids.shape[1]
    sm = (ids >= START_MIN) & (ids <= START_MAX)
    em = (ids >= END_MIN) & (ids <= END_MAX)
    s_cum = _incl_cumsum_lanes(sm.astype(jnp.int32))
    e_cum = _incl_cumsum_lanes(em.astype(jnp.int32))

    seg = lax.broadcasted_iota(jnp.int32, (N_SEG, s), 0)
    maskf = jnp.where(
        (s_cum == seg + 1) & (e_cum == seg) & (~sm) & (~em), 1.0, 0.0)

    sums = lax.dot_general(
        maskf.astype(jnp.bfloat16), x_ref[0].astype(jnp.bfloat16),
        dimension_numbers=(((1,), (0,)), ((), ())),
        preferred_element_type=jnp.float32,
    )
    counts = jnp.sum(maskf, axis=1, keepdims=True)
    o_ref[0] = sums / counts


def _tc_means(x, input_ids):
    b, s, d = x.shape
    ids3 = input_ids.reshape(b, 1, s)
    return pl.pallas_call(
        _tc_mean_kernel,
        grid=(b,),
        in_specs=[
            pl.BlockSpec((1, 1, s), lambda i: (i, 0, 0)),
            pl.BlockSpec((1, s, d), lambda i: (i, 0, 0)),
        ],
        out_specs=pl.BlockSpec((1, N_SEG, d), lambda i: (i, 0, 0)),
        out_shape=jax.ShapeDtypeStruct((b, N_SEG, d), jnp.float32),
        cost_estimate=pl.CostEstimate(
            flops=2 * b * N_SEG * s * d,
            bytes_accessed=b * s * d * 4,
            transcendentals=0,
        ),
    )(ids3, x)


def kernel(x, attention_mask, input_ids):
    del attention_mask
    b = x.shape[0]
    h = b // 2
    xx0, xx1 = _sc_gather(x, input_ids)
    m0 = _tc_means(x[:h], input_ids[:h])
    m1 = _tc_means(x[h:], input_ids[h:])
    xx2 = jnp.concatenate([m0, m1], axis=0)
    return jnp.concatenate([xx0, xx1, xx2], axis=-1)


# speedup vs baseline: 1.9074x; 1.9074x over previous
import functools

import jax
import jax.numpy as jnp
from jax import lax
from jax.experimental import pallas as pl
from jax.experimental.pallas import tpu as pltpu
from jax.experimental.pallas import tpu_sc as plsc

START_MIN, START_MAX = 1, 2
END_MIN, END_MAX = 3, 4
N_SEG = 16
LANES = 16


def _sc_gather_kernel(x2d, ids, out01, ids_v, idx_v, rows_v, tmp_v, sem,
                      *, b_total, s_len, d):
    wid = lax.axis_index("s")
    b = wid % b_total
    which = wid // b_total

    @pl.when(which < 2)
    def _():
        pltpu.sync_copy(ids.at[b], ids_v)
        lo = jnp.where(which == 0, START_MIN, END_MIN)
        hi = jnp.where(which == 0, START_MAX, END_MAX)
        lane = lax.iota(jnp.int32, LANES)

        def body(ci, cnt):
            v = ids_v[pl.ds(ci * LANES, LANES)]
            m = (v >= lo) & (v <= hi)

            def matched(cnt):
                cur = jnp.where(m, 1, 0)
                for sh in (1, 2, 4, 8):
                    tmp_v[...] = cur
                    g = plsc.load_gather(tmp_v, [jnp.maximum(lane - sh, 0)])
                    cur = cur + jnp.where(lane >= sh, g, 0)
                ranks = jnp.clip(cnt + cur - 1, 0, N_SEG - 1)
                pos = b * s_len + ci * LANES + lane
                plsc.store_scatter(idx_v, [ranks], pos, mask=m)
                return cnt + plsc.all_reduce_population_count(m)

            return lax.cond(jnp.any(m), matched, lambda c: c, cnt)

        lax.fori_loop(0, s_len // LANES, body,
                      jnp.zeros((LANES,), jnp.int32))
        pltpu.async_copy(x2d.at[idx_v], rows_v, sem).wait()

        @pl.when(which == 0)
        def _():
            pltpu.sync_copy(rows_v, out01.at[b, :, pl.ds(0, d)])

        @pl.when(which == 1)
        def _():
            pltpu.sync_copy(rows_v, out01.at[b, :, pl.ds(d, d)])


def _sc_gather(x, input_ids):
    b, s, d = x.shape
    x2d = x.reshape(b * s, d)
    mesh = plsc.VectorSubcoreMesh(
        core_axis_name="c", subcore_axis_name="s", num_cores=1)
    kern = functools.partial(
        pl.kernel,
        out_type=jax.ShapeDtypeStruct((b, N_SEG, 2 * d), jnp.float32),
        mesh=mesh,
        scratch_types=[
            pltpu.VMEM((s,), jnp.int32),
            pltpu.VMEM((N_SEG,), jnp.int32),
            pltpu.VMEM((N_SEG, d), jnp.float32),
            pltpu.VMEM((LANES,), jnp.int32),
            pltpu.SemaphoreType.DMA,
        ],
        compiler_params=pltpu.CompilerParams(
            needs_layout_passes=False, skip_device_barrier=True),
        cost_estimate=pl.CostEstimate(
            flops=0,
            bytes_accessed=2 * b * N_SEG * d * 4 + b * s * 4,
            transcendentals=0,
        ),
    )(functools.partial(_sc_gather_kernel, b_total=b, s_len=s, d=d))
    return kern(x2d, input_ids)


def _incl_cumsum_lanes(a):
    s = a.shape[1]
    sh = 1
    while sh < s:
        shifted = jnp.concatenate(
            [jnp.zeros((1, sh), a.dtype), a[:, : s - sh]], axis=1)
        a = a + shifted
        sh *= 2
    return a


def _tc_mean_kernel(ids_ref, x_ref, o_ref):
    ids = ids_ref[0]
    s = ids.shape[1]
    sm = (ids >= START_MIN) & (ids <= START_MAX)
    em = (ids >= END_MIN) & (ids <= END_MAX)
    s_cum = _incl_cumsum_lanes(sm.astype(jnp.int32))
    e_cum = _incl_cumsum_lanes(em.astype(jnp.int32))

    seg = lax.broadcasted_iota(jnp.int32, (N_SEG, s), 0)
    maskf = jnp.where(
        (s_cum == seg + 1) & (e_cum == seg) & (~sm) & (~em), 1.0, 0.0)

    sums = lax.dot_general(
        maskf.astype(jnp.bfloat16), x_ref[0].astype(jnp.bfloat16),
        dimension_numbers=(((1,), (0,)), ((), ())),
        preferred_element_type=jnp.float32,
    )
    counts = jnp.sum(maskf, axis=1, keepdims=True)
    o_ref[0] = sums / counts


def _tc_means(x, input_ids):
    b, s, d = x.shape
    ids3 = input_ids.reshape(b, 1, s)
    return pl.pallas_call(
        _tc_mean_kernel,
        grid=(b,),
        in_specs=[
            pl.BlockSpec((1, 1, s), lambda i: (i, 0, 0)),
            pl.BlockSpec((1, s, d), lambda i: (i, 0, 0)),
        ],
        out_specs=pl.BlockSpec((1, N_SEG, d), lambda i: (i, 0, 0)),
        out_shape=jax.ShapeDtypeStruct((b, N_SEG, d), jnp.float32),
        cost_estimate=pl.CostEstimate(
            flops=2 * b * N_SEG * s * d,
            bytes_accessed=b * s * d * 4,
            transcendentals=0,
        ),
    )(ids3, x)


def kernel(x, attention_mask, input_ids):
    del attention_mask
    xx01 = _sc_gather(x, input_ids)
    xx2 = _tc_means(x, input_ids)
    return jnp.concatenate([xx01, xx2], axis=-1)


# speedup vs baseline: 2.9026x; 1.5217x over previous
import jax
import jax.numpy as jnp
from jax import lax
from jax.experimental import pallas as pl

START_MIN, START_MAX = 1, 2
END_MIN, END_MAX = 3, 4
N_SEG = 16


def _incl_cumsum_lanes(a):
    s = a.shape[1]
    sh = 1
    while sh < s:
        shifted = jnp.concatenate(
            [jnp.zeros((1, sh), a.dtype), a[:, : s - sh]], axis=1)
        a = a + shifted
        sh *= 2
    return a


def _row_kernel(ids_ref, x_ref, o_ref):
    ids = ids_ref[0]
    s = ids.shape[1]
    sm = (ids >= START_MIN) & (ids <= START_MAX)
    em = (ids >= END_MIN) & (ids <= END_MAX)
    s_cum = _incl_cumsum_lanes(sm.astype(jnp.int32))
    e_cum = _incl_cumsum_lanes(em.astype(jnp.int32))

    r = lax.broadcasted_iota(jnp.int32, (3 * N_SEG, s), 0)
    seg = r // 3
    c = r % 3
    s_eq = s_cum == seg + 1
    is_start = jnp.where(sm & s_eq, 1.0, 0.0)
    is_end = jnp.where(em & (e_cum == seg + 1), 1.0, 0.0)
    is_interior = jnp.where(s_eq & (e_cum == seg) & (~sm) & (~em), 1.0, 0.0)
    maskf = jnp.where(c == 0, is_start,
                      jnp.where(c == 1, is_end, is_interior))

    sums = lax.dot_general(
        maskf.astype(jnp.bfloat16), x_ref[0].astype(jnp.bfloat16),
        dimension_numbers=(((1,), (0,)), ((), ())),
        preferred_element_type=jnp.float32,
    )
    counts = jnp.sum(maskf, axis=1, keepdims=True)
    cl = lax.broadcasted_iota(jnp.int32, (3 * N_SEG, 1), 0)
    div = jnp.where(cl % 3 == 2, counts, 1.0)
    o_ref[0] = sums / div


def kernel(x, attention_mask, input_ids):
    del attention_mask
    b, s, d = x.shape
    ids3 = input_ids.reshape(b, 1, s)
    out = pl.pallas_call(
        _row_kernel,
        grid=(b,),
        in_specs=[
            pl.BlockSpec((1, 1, s), lambda i: (i, 0, 0)),
            pl.BlockSpec((1, s, d), lambda i: (i, 0, 0)),
        ],
        out_specs=pl.BlockSpec((1, 3 * N_SEG, d), lambda i: (i, 0, 0)),
        out_shape=jax.ShapeDtypeStruct((b, 3 * N_SEG, d), jnp.float32),
    )(ids3, x)
    return out.reshape(b, N_SEG, 3 * d)
